# baseline (speedup 1.0000x reference)
import functools

import jax
import jax.numpy as jnp
from jax import lax
from jax.experimental import pallas as pl
from jax.experimental.pallas import tpu as pltpu
from jax.experimental.pallas import tpu_sc as plsc

NC = 2
NS = 16
NW = NC * NS
L = 16
CHUNK = 128
KS = 1
PACK_B = 14
PACK_M = (1 << PACK_B) - 1


def _mesh():
    return plsc.VectorSubcoreMesh(
        core_axis_name="c", subcore_axis_name="s",
        num_cores=NC, num_subcores=NS)


def _make_deg_kernel(n_pad, kc):
    rows_per_tile = n_pad // NS

    @functools.partial(
        pl.kernel,
        out_type=jax.ShapeDtypeStruct((NC, n_pad), jnp.float32),
        mesh=_mesh(),
        scratch_types=[
            pltpu.VMEM((kc, CHUNK), jnp.int32),
            pltpu.VMEM((CHUNK,), jnp.float32),
            pltpu.VMEM((rows_per_tile,), jnp.float32),
            pltpu.VMEM_SHARED((n_pad,), jnp.float32),
        ],
    )
    def deg_kernel(dst_hbm, out_hbm, idx_v, ones_v, zbuf, deg_sh):
        c = lax.axis_index("c")
        s = lax.axis_index("s")
        wid = c * NS + s
        for i in range(CHUNK // L):
            ones_v[pl.ds(i * L, L)] = jnp.ones((L,), jnp.float32)

        def zfill(i, carry):
            zbuf[pl.ds(i * L, L)] = jnp.zeros((L,), jnp.float32)
            return carry

        lax.fori_loop(0, rows_per_tile // L, zfill, 0)
        pltpu.sync_copy(zbuf, deg_sh.at[pl.ds(s * rows_per_tile, rows_per_tile)])
        pltpu.sync_copy(dst_hbm.at[wid], idx_v)
        plsc.subcore_barrier()

        def body(j, carry):
            pltpu.sync_copy(ones_v, deg_sh.at[idx_v.at[j]], add=True)
            return carry

        lax.fori_loop(0, kc, body, 0)
        plsc.subcore_barrier()
        pltpu.sync_copy(
            deg_sh.at[pl.ds(s * rows_per_tile, rows_per_tile)],
            out_hbm.at[c, pl.ds(s * rows_per_tile, rows_per_tile)])

    return deg_kernel


def _make_edge_kernel(n_pad, kc, d):
    rows_per_tile = n_pad // NS

    @functools.partial(
        pl.kernel,
        out_type=jax.ShapeDtypeStruct((NC, n_pad, d), jnp.float32),
        mesh=_mesh(),
        scratch_types=[
            pltpu.VMEM((kc, CHUNK), jnp.int32),
            pltpu.VMEM((2, KS * CHUNK), jnp.int32),
            pltpu.VMEM((KS * CHUNK, d), jnp.float32),
            pltpu.VMEM_SHARED((n_pad, d), jnp.float32),
            pltpu.SemaphoreType.DMA,
        ],
    )
    def edge_kernel(y_hbm, edges_hbm, out_hbm,
                    pidx_v, uidx_v, rows_v, s_sh, sem):
        c = lax.axis_index("c")
        s = lax.axis_index("s")
        wid = c * NS + s

        def zfill(i, carry):
            r = i // (d // L)
            k = (i % (d // L)) * L
            rows_v[r, pl.ds(k, L)] = jnp.zeros((L,), jnp.float32)
            return carry

        lax.fori_loop(0, CHUNK * (d // L), zfill, 0)
        for t in range(rows_per_tile // CHUNK):
            pltpu.sync_copy(
                rows_v.at[pl.ds(0, CHUNK)],
                s_sh.at[pl.ds(s * rows_per_tile + t * CHUNK, CHUNK)])
        pltpu.sync_copy(edges_hbm.at[wid], pidx_v)
        plsc.subcore_barrier()

        def body(g, carry):
            def unpack(i, carry2):
                t = i // (CHUNK // L)
                k = (i % (CHUNK // L)) * L
                v = pidx_v[g * KS + t, pl.ds(k, L)]
                uidx_v[0, pl.ds(t * CHUNK + k, L)] = jnp.bitwise_and(
                    v, PACK_M)
                uidx_v[1, pl.ds(t * CHUNK + k, L)] = jnp.right_shift(
                    v, PACK_B)
                return carry2

            lax.fori_loop(0, KS * (CHUNK // L), unpack, 0)
            pltpu.async_copy(y_hbm.at[uidx_v.at[0]], rows_v, sem).wait()
            pltpu.sync_copy(rows_v, s_sh.at[uidx_v.at[1]], add=True)
            return carry

        lax.fori_loop(0, kc // KS, body, 0)
        plsc.subcore_barrier()
        for t in range(rows_per_tile // CHUNK):
            base = s * rows_per_tile + t * CHUNK
            pltpu.sync_copy(s_sh.at[pl.ds(base, CHUNK)],
                            out_hbm.at[c, pl.ds(base, CHUNK)])

    return edge_kernel


def _scale_body(degp_ref, x_ref, y_ref):
    deg = degp_ref[0] + degp_ref[1] + 1.0
    dis = lax.rsqrt(deg)
    y_ref[...] = x_ref[...] * dis


def _out_body(degp_ref, s_ref, x_ref, wl_ref, bl_ref, wh_ref, bh_ref,
              lo_ref, hi_ref):
    deg = degp_ref[0] + degp_ref[1] + 1.0
    dis = lax.rsqrt(deg)
    stot = s_ref[0] + s_ref[1]
    agg = dis * stot + (dis * dis) * x_ref[...]
    lo_ref[...] = jnp.dot(agg, wl_ref[...],
                          preferred_element_type=jnp.float32) + bl_ref[...]
    hi_ref[...] = jnp.dot(agg, wh_ref[...],
                          preferred_element_type=jnp.float32) + bh_ref[...]


def kernel(x, edge_index, W_low, b_low, W_high, b_high):
    n, d = x.shape
    hid = W_low.shape[1]
    e = edge_index.shape[1]

    tile_rows = NS * CHUNK
    n_pad = ((n + 1 + tile_rows - 1) // tile_rows) * tile_rows
    per_round = NW * CHUNK
    kc = (e + per_round - 1) // per_round
    kc = kc + (kc % 2)
    e_pad = kc * per_round

    src = edge_index[0].astype(jnp.int32)
    dst = edge_index[1].astype(jnp.int32)
    pad = e_pad - e
    src_pad = jnp.concatenate(
        [src, jnp.zeros((pad,), jnp.int32)]).reshape(NW, kc, CHUNK)
    dst_pad = jnp.concatenate(
        [dst, jnp.full((pad,), n, jnp.int32)]).reshape(NW, kc, CHUNK)
    edges = src_pad | (dst_pad << PACK_B)
    x_pad = jnp.pad(x, ((0, n_pad - n), (0, 0)))

    degp = _make_deg_kernel(n_pad, kc)(dst_pad)
    degp3 = degp.reshape(NC, n_pad, 1)

    blk = 512
    grid = n_pad // blk
    y = pl.pallas_call(
        _scale_body,
        grid=(grid,),
        in_specs=[
            pl.BlockSpec((NC, blk, 1), lambda i: (0, i, 0)),
            pl.BlockSpec((blk, d), lambda i: (i, 0)),
        ],
        out_specs=pl.BlockSpec((blk, d), lambda i: (i, 0)),
        out_shape=jax.ShapeDtypeStruct((n_pad, d), jnp.float32),
    )(degp3, x_pad)

    S = _make_edge_kernel(n_pad, kc, d)(y, edges)

    lo_pad, hi_pad = pl.pallas_call(
        _out_body,
        grid=(grid,),
        in_specs=[
            pl.BlockSpec((NC, blk, 1), lambda i: (0, i, 0)),
            pl.BlockSpec((NC, blk, d), lambda i: (0, i, 0)),
            pl.BlockSpec((blk, d), lambda i: (i, 0)),
            pl.BlockSpec((d, hid), lambda i: (0, 0)),
            pl.BlockSpec((1, hid), lambda i: (0, 0)),
            pl.BlockSpec((d, hid), lambda i: (0, 0)),
            pl.BlockSpec((1, hid), lambda i: (0, 0)),
        ],
        out_specs=[
            pl.BlockSpec((blk, hid), lambda i: (i, 0)),
            pl.BlockSpec((blk, hid), lambda i: (i, 0)),
        ],
        out_shape=[
            jax.ShapeDtypeStruct((n_pad, hid), jnp.float32),
            jax.ShapeDtypeStruct((n_pad, hid), jnp.float32),
        ],
    )(degp3, S, x_pad, W_low, b_low.reshape(1, hid),
      W_high, b_high.reshape(1, hid))

    return (lo_pad[:n], hi_pad[:n])


# speedup vs baseline: 1.6105x; 1.6105x over previous
import functools

import jax
import jax.numpy as jnp
from jax import lax
from jax.experimental import pallas as pl
from jax.experimental.pallas import tpu as pltpu
from jax.experimental.pallas import tpu_sc as plsc

NC = 2
NS = 16
NW = NC * NS
L = 16
CHUNK = 128
FAST_SHARE = 0.65


def _mesh():
    return plsc.VectorSubcoreMesh(
        core_axis_name="c", subcore_axis_name="s",
        num_cores=NC, num_subcores=NS)


def _make_deg_kernel(n_pad, kc):
    rows_per_tile = n_pad // NS

    @functools.partial(
        pl.kernel,
        out_type=jax.ShapeDtypeStruct((NC, n_pad), jnp.float32),
        mesh=_mesh(),
        scratch_types=[
            pltpu.VMEM((kc, CHUNK), jnp.int32),
            pltpu.VMEM((CHUNK,), jnp.float32),
            pltpu.VMEM((rows_per_tile,), jnp.float32),
            pltpu.VMEM_SHARED((n_pad,), jnp.float32),
        ],
    )
    def deg_kernel(dst_hbm, out_hbm, idx_v, ones_v, zbuf, deg_sh):
        c = lax.axis_index("c")
        s = lax.axis_index("s")
        wid = c * NS + s
        for i in range(CHUNK // L):
            ones_v[pl.ds(i * L, L)] = jnp.ones((L,), jnp.float32)

        def zfill(i, carry):
            zbuf[pl.ds(i * L, L)] = jnp.zeros((L,), jnp.float32)
            return carry

        lax.fori_loop(0, rows_per_tile // L, zfill, 0)
        pltpu.sync_copy(zbuf, deg_sh.at[pl.ds(s * rows_per_tile, rows_per_tile)])
        pltpu.sync_copy(dst_hbm.at[wid], idx_v)
        plsc.subcore_barrier()

        def body(j, carry):
            pltpu.sync_copy(ones_v, deg_sh.at[idx_v.at[j]], add=True)
            return carry

        lax.fori_loop(0, kc, body, 0)
        plsc.subcore_barrier()
        pltpu.sync_copy(
            deg_sh.at[pl.ds(s * rows_per_tile, rows_per_tile)],
            out_hbm.at[c, pl.ds(s * rows_per_tile, rows_per_tile)])

    return deg_kernel


def _make_edge_kernel(n_pad, kc0, kc1, d):
    rows_per_tile = n_pad // NS
    kc_max = max(kc0, kc1)

    @functools.partial(
        pl.kernel,
        out_type=jax.ShapeDtypeStruct((NC, n_pad, d), jnp.float32),
        mesh=_mesh(),
        scratch_types=[
            pltpu.VMEM((kc_max, CHUNK), jnp.int32),
            pltpu.VMEM((kc_max, CHUNK), jnp.int32),
            pltpu.VMEM((CHUNK, d), jnp.float32),
            pltpu.VMEM_SHARED((n_pad, d), jnp.float32),
            pltpu.SemaphoreType.DMA,
        ],
    )
    def edge_kernel(y_hbm, src_hbm, dst_hbm, out_hbm,
                    src_v, dst_v, rows_v, s_sh, sem):
        c = lax.axis_index("c")
        s = lax.axis_index("s")
        wid = c * NS + s
        kc_c = jnp.where(c == 0, kc0, kc1)

        def zfill(i, carry):
            r = i // (d // L)
            k = (i % (d // L)) * L
            rows_v[r, pl.ds(k, L)] = jnp.zeros((L,), jnp.float32)
            return carry

        lax.fori_loop(0, CHUNK * (d // L), zfill, 0)
        for t in range(rows_per_tile // CHUNK):
            pltpu.sync_copy(
                rows_v,
                s_sh.at[pl.ds(s * rows_per_tile + t * CHUNK, CHUNK)])
        pltpu.sync_copy(src_hbm.at[wid], src_v)
        pltpu.sync_copy(dst_hbm.at[wid], dst_v)
        plsc.subcore_barrier()

        def body(j, carry):
            pltpu.async_copy(y_hbm.at[src_v.at[j]], rows_v, sem).wait()
            pltpu.sync_copy(rows_v, s_sh.at[dst_v.at[j]], add=True)
            return carry

        lax.fori_loop(0, kc_c, body, 0)
        plsc.subcore_barrier()
        for t in range(rows_per_tile // CHUNK):
            base = s * rows_per_tile + t * CHUNK
            pltpu.sync_copy(s_sh.at[pl.ds(base, CHUNK)],
                            out_hbm.at[c, pl.ds(base, CHUNK)])

    return edge_kernel


def _scale_body(degp_ref, x_ref, y_ref):
    deg = degp_ref[0] + degp_ref[1] + 1.0
    dis = lax.rsqrt(deg)
    y_ref[...] = x_ref[...] * dis


def _out_body(degp_ref, s_ref, x_ref, wl_ref, bl_ref, wh_ref, bh_ref,
              lo_ref, hi_ref):
    deg = degp_ref[0] + degp_ref[1] + 1.0
    dis = lax.rsqrt(deg)
    stot = s_ref[0] + s_ref[1]
    agg = dis * stot + (dis * dis) * x_ref[...]
    lo_ref[...] = jnp.dot(agg, wl_ref[...],
                          preferred_element_type=jnp.float32) + bl_ref[...]
    hi_ref[...] = jnp.dot(agg, wh_ref[...],
                          preferred_element_type=jnp.float32) + bh_ref[...]


def kernel(x, edge_index, W_low, b_low, W_high, b_high):
    n, d = x.shape
    hid = W_low.shape[1]
    e = edge_index.shape[1]

    tile_rows = NS * CHUNK
    n_pad = ((n + 1 + tile_rows - 1) // tile_rows) * tile_rows
    per_round = NW * CHUNK
    kc = (e + per_round - 1) // per_round
    e_pad = kc * per_round

    src = edge_index[0].astype(jnp.int32)
    dst = edge_index[1].astype(jnp.int32)
    pad = e_pad - e
    src_pad = jnp.concatenate(
        [src, jnp.zeros((pad,), jnp.int32)]).reshape(NW, kc, CHUNK)
    dst_pad = jnp.concatenate(
        [dst, jnp.full((pad,), n, jnp.int32)]).reshape(NW, kc, CHUNK)
    x_pad = jnp.pad(x, ((0, n_pad - n), (0, 0)))

    kc_all = 2 * kc
    kc0 = max(2, min(kc_all - 2, round(kc_all * FAST_SHARE)))
    kc1 = kc_all - kc0
    cap0 = NS * kc0 * CHUNK
    e_pad2 = (kc0 + kc1) * NS * CHUNK

    def split_pad(v, fill):
        flat = jnp.concatenate([v, jnp.full((e_pad2 - e,), fill, jnp.int32)])
        b0 = flat[:cap0].reshape(NS, kc0, CHUNK)
        b1 = flat[cap0:].reshape(NS, kc1, CHUNK)
        kc_max = max(kc0, kc1)
        b0 = jnp.pad(b0, ((0, 0), (0, kc_max - kc0), (0, 0)),
                     constant_values=fill)
        b1 = jnp.pad(b1, ((0, 0), (0, kc_max - kc1), (0, 0)),
                     constant_values=fill)
        return jnp.concatenate([b0, b1], axis=0)

    src_sp = split_pad(src, 0)
    dst_sp = split_pad(dst, n)

    degp = _make_deg_kernel(n_pad, kc)(dst_pad)
    degp3 = degp.reshape(NC, n_pad, 1)

    blk = 512
    grid = n_pad // blk
    y = pl.pallas_call(
        _scale_body,
        grid=(grid,),
        in_specs=[
            pl.BlockSpec((NC, blk, 1), lambda i: (0, i, 0)),
            pl.BlockSpec((blk, d), lambda i: (i, 0)),
        ],
        out_specs=pl.BlockSpec((blk, d), lambda i: (i, 0)),
        out_shape=jax.ShapeDtypeStruct((n_pad, d), jnp.float32),
    )(degp3, x_pad)

    S = _make_edge_kernel(n_pad, kc0, kc1, d)(y, src_sp, dst_sp)

    lo_pad, hi_pad = pl.pallas_call(
        _out_body,
        grid=(grid,),
        in_specs=[
            pl.BlockSpec((NC, blk, 1), lambda i: (0, i, 0)),
            pl.BlockSpec((NC, blk, d), lambda i: (0, i, 0)),
            pl.BlockSpec((blk, d), lambda i: (i, 0)),
            pl.BlockSpec((d, hid), lambda i: (0, 0)),
            pl.BlockSpec((1, hid), lambda i: (0, 0)),
            pl.BlockSpec((d, hid), lambda i: (0, 0)),
            pl.BlockSpec((1, hid), lambda i: (0, 0)),
        ],
        out_specs=[
            pl.BlockSpec((blk, hid), lambda i: (i, 0)),
            pl.BlockSpec((blk, hid), lambda i: (i, 0)),
        ],
        out_shape=[
            jax.ShapeDtypeStruct((n_pad, hid), jnp.float32),
            jax.ShapeDtypeStruct((n_pad, hid), jnp.float32),
        ],
    )(degp3, S, x_pad, W_low, b_low.reshape(1, hid),
      W_high, b_high.reshape(1, hid))

    return (lo_pad[:n], hi_pad[:n])
